# baseline (speedup 1.0000x reference)
import functools

import numpy as np
import jax
import jax.numpy as jnp
from jax import lax
from jax.experimental import pallas as pl
from jax.experimental.pallas import tpu as pltpu
from jax.experimental.pallas import tpu_sc as plsc

B = 4096
NCAT = 26
NCONT = 13
VOCAB = 100000
ED = 32
NEMB = NCAT * ED
L1 = 512
L2 = 256
NCLS = 2
EPS = 1e-5

_NC = 2
_NS = 16
_NW = _NC * _NS
_BPW = B // _NW
_RPW = _BPW * NCAT
_CHUNK = 128
_NCHUNK = _RPW // _CHUNK


def _sc_gather_body(xcat_hbm, foff_hbm, table_hbm, out_hbm,
                    xcat_v, foff_v, idx_v, rows_v, sem):
    wid = lax.axis_index("s") * _NC + lax.axis_index("c")
    base = wid * _RPW
    pltpu.sync_copy(xcat_hbm.at[pl.ds(base, _RPW)], xcat_v)
    pltpu.sync_copy(foff_hbm, foff_v)
    for t in range(_RPW // 16):
        vals = xcat_v[pl.ds(t * 16, 16)]
        fo = foff_v[pl.ds(t * 16, 16)]
        idx_v[t // 8, pl.ds((t % 8) * 16, 16)] = fo + vals.astype(jnp.int32)
    for half in range(2):
        descs = [
            pltpu.make_async_copy(
                table_hbm.at[idx_v.at[c]],
                rows_v.at[pl.ds(c * _CHUNK, _CHUNK), :],
                sem,
            )
            for c in range(half * 13, half * 13 + 13)
        ]
        for d in descs:
            d.start()
        for d in descs:
            d.wait()
    pltpu.sync_copy(rows_v, out_hbm.at[pl.ds(base, _RPW), :])


def _make_sc_gather():
    return functools.partial(
        pl.kernel,
        out_type=jax.ShapeDtypeStruct((B * NCAT, ED), jnp.float32),
        mesh=plsc.VectorSubcoreMesh(core_axis_name="c", subcore_axis_name="s",
                                    num_cores=_NC, num_subcores=_NS),
        scratch_types=[
            pltpu.VMEM((_RPW,), jnp.float32),
            pltpu.VMEM((_RPW,), jnp.int32),
            pltpu.VMEM((_NCHUNK, _CHUNK), jnp.int32),
            pltpu.VMEM((_RPW, ED), jnp.float32),
            pltpu.SemaphoreType.DMA,
        ],
    )(_sc_gather_body)


def _bn_cols(v, g, b):
    m = jnp.mean(v, axis=0, keepdims=True)
    vm = v - m
    var = jnp.mean(vm * vm, axis=0, keepdims=True)
    return vm * lax.rsqrt(var + EPS) * g + b


def _mlp_body(x1_ref, xc_ref, w1a_ref, w1b_ref, b1_ref, w2_ref, b2_ref,
              w3_ref, b3_ref, g1_ref, be1_ref, g2_ref, be2_ref, g3_ref,
              be3_ref, out_ref):
    x2 = _bn_cols(xc_ref[:], g1_ref[:], be1_ref[:])
    h = jnp.dot(x1_ref[:], w1a_ref[:], preferred_element_type=jnp.float32)
    h = h + jnp.dot(x2, w1b_ref[:], preferred_element_type=jnp.float32)
    h = jnp.maximum(h + b1_ref[:], 0.0)
    h = _bn_cols(h, g2_ref[:], be2_ref[:])
    h = jnp.dot(h, w2_ref[:], preferred_element_type=jnp.float32)
    h = jnp.maximum(h + b2_ref[:], 0.0)
    h = _bn_cols(h, g3_ref[:], be3_ref[:])
    out_ref[:] = (
        jnp.dot(h, w3_ref[:], preferred_element_type=jnp.float32) + b3_ref[:]
    )


_mlp = pl.pallas_call(
    _mlp_body,
    out_shape=jax.ShapeDtypeStruct((B, NCLS), jnp.float32),
)


def kernel(x, emb_tables, W1, b1, W2, b2, W3, b3, g1, be1, g2, be2, g3, be3):
    x_cat = x[:, :NCAT].astype(jnp.int32)
    table = emb_tables.reshape(NCAT * VOCAB, ED)
    flat_idx = (jnp.arange(NCAT, dtype=jnp.int32)[None, :] * VOCAB + x_cat)
    x1 = jnp.take(table, flat_idx.reshape(-1), axis=0).reshape(B, NEMB)
    xc = x[:, NCAT:]
    return _mlp(
        x1, xc, W1[:NEMB], W1[NEMB:], b1.reshape(1, L1), W2,
        b2.reshape(1, L2), W3, b3.reshape(1, NCLS), g1.reshape(1, NCONT),
        be1.reshape(1, NCONT), g2.reshape(1, L1), be2.reshape(1, L1),
        g3.reshape(1, L2), be3.reshape(1, L2),
    )


# speedup vs baseline: 50.2573x; 50.2573x over previous
import functools

import numpy as np
import jax
import jax.numpy as jnp
from jax import lax
from jax.experimental import pallas as pl
from jax.experimental.pallas import tpu as pltpu
from jax.experimental.pallas import tpu_sc as plsc

B = 4096
NCAT = 26
NCONT = 13
VOCAB = 100000
ED = 32
NEMB = NCAT * ED
L1 = 512
L2 = 256
NCLS = 2
EPS = 1e-5

_NC = 2
_NS = 16
_NW = _NC * _NS


def _sc_gather_body(t3_hbm, xcatt_hbm, out_hbm, row_v, xf_v, res_v, sem):
    w = lax.axis_index("s") * _NC + lax.axis_index("c")

    def field_body(c, carry):
        pltpu.sync_copy(t3_hbm.at[c, w, :], row_v)
        pltpu.sync_copy(xcatt_hbm.at[c, :], xf_v)

        def group_body(m, carry2):
            for u in range(16):
                off = m * 256 + u * 16
                vi = xf_v[pl.ds(off, 16)].astype(jnp.int32)
                res_v[pl.ds(off, 16)] = plsc.load_gather(row_v, [vi])
            return carry2

        lax.fori_loop(0, B // 256, group_body, 0)
        pltpu.sync_copy(res_v, out_hbm.at[c * _NW + w, :])
        return carry

    lax.fori_loop(0, NCAT, field_body, 0)


def _make_sc_gather():
    return functools.partial(
        pl.kernel,
        out_type=jax.ShapeDtypeStruct((NEMB, B), jnp.float32),
        mesh=plsc.VectorSubcoreMesh(core_axis_name="c", subcore_axis_name="s",
                                    num_cores=_NC, num_subcores=_NS),
        scratch_types=[
            pltpu.VMEM((VOCAB,), jnp.float32),
            pltpu.VMEM((B,), jnp.float32),
            pltpu.VMEM((B,), jnp.float32),
            pltpu.SemaphoreType.DMA,
        ],
        compiler_params=pltpu.CompilerParams(needs_layout_passes=False),
    )(_sc_gather_body)


def _bn_cols(v, g, b):
    m = jnp.mean(v, axis=0, keepdims=True)
    vm = v - m
    var = jnp.mean(vm * vm, axis=0, keepdims=True)
    return vm * lax.rsqrt(var + EPS) * g + b


def _mlp_body(x1t_ref, xc_ref, w1a_ref, w1b_ref, b1_ref, w2_ref, b2_ref,
              w3_ref, b3_ref, g1_ref, be1_ref, g2_ref, be2_ref, g3_ref,
              be3_ref, out_ref):
    x2 = _bn_cols(xc_ref[:], g1_ref[:], be1_ref[:])
    h = lax.dot_general(x1t_ref[:], w1a_ref[:], (((0,), (0,)), ((), ())),
                        preferred_element_type=jnp.float32)
    h = h + jnp.dot(x2, w1b_ref[:], preferred_element_type=jnp.float32)
    h = jnp.maximum(h + b1_ref[:], 0.0)
    h = _bn_cols(h, g2_ref[:], be2_ref[:])
    h = jnp.dot(h, w2_ref[:], preferred_element_type=jnp.float32)
    h = jnp.maximum(h + b2_ref[:], 0.0)
    h = _bn_cols(h, g3_ref[:], be3_ref[:])
    out_ref[:] = (
        jnp.dot(h, w3_ref[:], preferred_element_type=jnp.float32) + b3_ref[:]
    )


_mlp = pl.pallas_call(
    _mlp_body,
    out_shape=jax.ShapeDtypeStruct((B, NCLS), jnp.float32),
)


def kernel(x, emb_tables, W1, b1, W2, b2, W3, b3, g1, be1, g2, be2, g3, be3):
    t3 = jnp.swapaxes(emb_tables, 1, 2)
    xcatt = x[:, :NCAT].T
    x1t = _make_sc_gather()(t3, xcatt)
    xc = x[:, NCAT:]
    return _mlp(
        x1t, xc, W1[:NEMB], W1[NEMB:], b1.reshape(1, L1), W2,
        b2.reshape(1, L2), W3, b3.reshape(1, NCLS), g1.reshape(1, NCONT),
        be1.reshape(1, NCONT), g2.reshape(1, L1), be2.reshape(1, L1),
        g3.reshape(1, L2), be3.reshape(1, L2),
    )
